# baseline (speedup 1.0000x reference)
import jax
import jax.numpy as jnp
from jax.experimental import pallas as pl
from jax.experimental.pallas import tpu as pltpu

EMBD = 128
B = 16384
R8 = B // 8
G = 4
CH = R8 // G


def _depie_body(u_hbm, t_ref, w_ref, b_ref, o_hbm, ubuf, obuf, isems, osems):
    i = pl.program_id(0)
    slot = jax.lax.rem(i, 2)
    nslot = jax.lax.rem(i + 1, 2)

    def in_copy(k, s):
        return pltpu.make_async_copy(
            u_hbm.at[pl.ds(k * CH, CH)], ubuf.at[s], isems.at[s]
        )

    def out_copy(k, s):
        return pltpu.make_async_copy(
            obuf.at[s], o_hbm.at[pl.ds(k * CH, CH)], osems.at[s]
        )

    @pl.when(i == 0)
    def _():
        in_copy(0, 0).start()

    @pl.when(i + 1 < G)
    def _():
        in_copy(i + 1, nslot).start()

    in_copy(i, slot).wait()

    @pl.when(i >= 2)
    def _():
        out_copy(i - 2, slot).wait()

    t = t_ref[...][:, :, None]
    coef = t * w_ref[...] + (b_ref[...] + 1.0)
    obuf[slot] = ubuf[slot] * coef

    out_copy(i, slot).start()

    @pl.when(i == G - 1)
    def _():
        out_copy(G - 2, nslot).wait()
        out_copy(G - 1, slot).wait()


@jax.jit
def _depie_tc(user3, td2, w3, b3):
    return pl.pallas_call(
        _depie_body,
        grid=(G,),
        in_specs=[
            pl.BlockSpec(memory_space=pltpu.MemorySpace.HBM),
            pl.BlockSpec((CH, 8), lambda i: (i, 0)),
            pl.BlockSpec((1, 1, EMBD), lambda i: (0, 0, 0)),
            pl.BlockSpec((1, 1, EMBD), lambda i: (0, 0, 0)),
        ],
        out_specs=pl.BlockSpec(memory_space=pltpu.MemorySpace.HBM),
        out_shape=jax.ShapeDtypeStruct((R8, 8, EMBD), jnp.float32),
        scratch_shapes=[
            pltpu.VMEM((2, CH, 8, EMBD), jnp.float32),
            pltpu.VMEM((2, CH, 8, EMBD), jnp.float32),
            pltpu.SemaphoreType.DMA((2,)),
            pltpu.SemaphoreType.DMA((2,)),
        ],
        compiler_params=pltpu.CompilerParams(
            dimension_semantics=("arbitrary",),
        ),
    )(user3, td2, w3, b3)


def kernel(user_embd, item_embd, timediffs, W_embd, b_embd):
    del item_embd
    user3 = user_embd.reshape(R8, 8, EMBD)
    td2 = timediffs[:, 0].reshape(R8, 8)
    w3 = W_embd.reshape(1, 1, EMBD)
    b3 = b_embd.reshape(1, 1, EMBD)
    return _depie_tc(user3, td2, w3, b3).reshape(B, EMBD)


# speedup vs baseline: 1.2100x; 1.2100x over previous
import jax
import jax.numpy as jnp
from jax.experimental import pallas as pl
from jax.experimental.pallas import tpu as pltpu

EMBD = 128
B = 16384
R8 = B // 8
BLOCK_G = 1024


def _depie_body(u_ref, t_ref, w_ref, b_ref, o_ref):
    t = t_ref[...][:, :, None]
    coef = t * w_ref[...] + (b_ref[...] + 1.0)
    o_ref[...] = u_ref[...] * coef


@jax.jit
def _depie_tc(user3, td2, w3, b3):
    grid = (R8 // BLOCK_G,)
    return pl.pallas_call(
        _depie_body,
        grid=grid,
        in_specs=[
            pl.BlockSpec((BLOCK_G, 8, EMBD), lambda i: (i, 0, 0)),
            pl.BlockSpec((BLOCK_G, 8), lambda i: (i, 0)),
            pl.BlockSpec((1, 1, EMBD), lambda i: (0, 0, 0)),
            pl.BlockSpec((1, 1, EMBD), lambda i: (0, 0, 0)),
        ],
        out_specs=pl.BlockSpec((BLOCK_G, 8, EMBD), lambda i: (i, 0, 0)),
        out_shape=jax.ShapeDtypeStruct((R8, 8, EMBD), jnp.float32),
        compiler_params=pltpu.CompilerParams(
            dimension_semantics=("arbitrary",),
        ),
    )(user3, td2, w3, b3)


def kernel(user_embd, item_embd, timediffs, W_embd, b_embd):
    del item_embd
    user3 = user_embd.reshape(R8, 8, EMBD)
    td2 = timediffs[:, 0].reshape(R8, 8)
    w3 = W_embd.reshape(1, 1, EMBD)
    b3 = b_embd.reshape(1, 1, EMBD)
    out3 = _depie_tc(user3, td2, w3, b3)
    return out3.reshape(B, EMBD)
